# Initial kernel scaffold; baseline (speedup 1.0000x reference)
import functools

import jax
import jax.numpy as jnp
from jax import lax
from jax.experimental import pallas as pl
from jax.experimental.pallas import tpu as pltpu
from jax.experimental.pallas import tpu_sc as plsc

N_NODES = 10000
D = 128
N_EDGES = 320000

NC = 2
NS = 16
NW = NC * NS
CHUNK = 128
NCH = -(-N_EDGES // (NW * CHUNK))
E_PAD = NW * CHUNK * NCH
EPW = NCH * CHUNK
RPT = 632
N_ACC = NS * RPT
CW = 16

_mesh = plsc.VectorSubcoreMesh(core_axis_name="c", subcore_axis_name="s")


def _sc_segment_sum(with_counts):
    out_type = [jax.ShapeDtypeStruct((NC * N_ACC, D), jnp.float32)]
    scratch = [
        pltpu.VMEM((NCH, CHUNK), jnp.int32),
        pltpu.VMEM((NCH, CHUNK), jnp.int32),
        pltpu.VMEM((CHUNK, D), jnp.float32),
        pltpu.VMEM_SHARED((N_ACC, D), jnp.float32),
        pltpu.SemaphoreType.DMA,
    ]
    if with_counts:
        out_type.append(jax.ShapeDtypeStruct((NC * N_ACC, CW), jnp.float32))
        scratch += [
            pltpu.VMEM((CHUNK, CW), jnp.float32),
            pltpu.VMEM_SHARED((N_ACC, CW), jnp.float32),
        ]

    def body(x_hbm, src_hbm, dst_hbm, zf_hbm, zc_hbm, ones_hbm,
             sum_out, *rest):
        if with_counts:
            cnt_out, srcv, dstv, rows, accf, sem, onesv, accc = rest
        else:
            (srcv, dstv, rows, accf, sem) = rest
        c = lax.axis_index("c")
        s = lax.axis_index("s")
        w = c * NS + s

        pltpu.sync_copy(zf_hbm.at[pl.ds(s * RPT, RPT)],
                        accf.at[pl.ds(s * RPT, RPT)])
        pltpu.sync_copy(src_hbm.at[pl.ds(w * NCH, NCH)], srcv)
        pltpu.sync_copy(dst_hbm.at[pl.ds(w * NCH, NCH)], dstv)
        if with_counts:
            pltpu.sync_copy(zc_hbm.at[pl.ds(s * RPT, RPT)],
                            accc.at[pl.ds(s * RPT, RPT)])
            pltpu.sync_copy(ones_hbm, onesv)
        plsc.subcore_barrier()

        @pl.loop(0, NCH)
        def _(j):
            pltpu.async_copy(x_hbm.at[srcv.at[j]], rows, sem).wait()
            pltpu.sync_copy(rows, accf.at[dstv.at[j]], add=True)
            if with_counts:
                pltpu.sync_copy(onesv, accc.at[dstv.at[j]], add=True)

        plsc.subcore_barrier()
        pltpu.sync_copy(accf.at[pl.ds(s * RPT, RPT)],
                        sum_out.at[pl.ds(c * N_ACC + s * RPT, RPT)])
        if with_counts:
            pltpu.sync_copy(accc.at[pl.ds(s * RPT, RPT)],
                            cnt_out.at[pl.ds(c * N_ACC + s * RPT, RPT)])

    return pl.kernel(body, out_type=out_type, mesh=_mesh,
                     scratch_types=scratch)


_sc_sum_counts = _sc_segment_sum(True)
_sc_sum = _sc_segment_sum(False)

BLK = 2000


def _dense_body(act, p0, p1, c0, c1, x, wl, wr, b, o):
    cnt = jnp.maximum(c0[:, 0:1] + c1[:, 0:1], 1.0)
    mean = (p0[...] + p1[...]) / cnt
    y = (jnp.dot(mean, wl[...], preferred_element_type=jnp.float32,
                 precision=lax.Precision.HIGHEST)
         + jnp.dot(x[...], wr[...], preferred_element_type=jnp.float32,
                   precision=lax.Precision.HIGHEST)
         + b[...])
    o[...] = jax.nn.relu(y) if act == "relu" else jax.nn.sigmoid(y)


def _dense(act):
    row_spec = pl.BlockSpec((BLK, D), lambda i: (i, 0))
    cnt_spec = pl.BlockSpec((BLK, CW), lambda i: (i, 0))
    full_spec = pl.BlockSpec((D, D), lambda i: (0, 0))
    b_spec = pl.BlockSpec((1, D), lambda i: (0, 0))
    return pl.pallas_call(
        functools.partial(_dense_body, act),
        grid=(N_NODES // BLK,),
        in_specs=[row_spec, row_spec, cnt_spec, cnt_spec, row_spec,
                  full_spec, full_spec, b_spec],
        out_specs=row_spec,
        out_shape=jax.ShapeDtypeStruct((N_NODES, D), jnp.float32),
    )


_dense_relu = _dense("relu")
_dense_sigmoid = _dense("sigmoid")


def kernel(x, edge_index, W1_l, W1_r, b1, W2_l, W2_r, b2):
    src = edge_index[0].astype(jnp.int32)
    dst = edge_index[1].astype(jnp.int32)
    pad = E_PAD - N_EDGES
    src_p = jnp.concatenate([src, jnp.zeros((pad,), jnp.int32)])
    dst_p = jnp.concatenate([dst, jnp.full((pad,), N_NODES, jnp.int32)])
    src_p = src_p.reshape(NW * NCH, CHUNK)
    dst_p = dst_p.reshape(NW * NCH, CHUNK)

    zf = jnp.zeros((N_ACC, D), jnp.float32)
    zc = jnp.zeros((N_ACC, CW), jnp.float32)
    ones = jnp.ones((CHUNK, CW), jnp.float32)

    def pad_rows(a):
        return jnp.concatenate(
            [a, jnp.zeros((N_ACC - a.shape[0], D), a.dtype)])

    x_pad = pad_rows(x)
    sums1, cnts = _sc_sum_counts(x_pad, src_p, dst_p, zf, zc, ones)
    c0 = cnts[:N_NODES]
    c1 = cnts[N_ACC:N_ACC + N_NODES]
    h = _dense_relu(sums1[:N_NODES], sums1[N_ACC:N_ACC + N_NODES],
                    c0, c1, x, W1_l.T, W1_r.T, b1.reshape(1, D))
    sums2 = _sc_sum(pad_rows(h), src_p, dst_p, zf, zc, ones)
    out = _dense_sigmoid(sums2[:N_NODES], sums2[N_ACC:N_ACC + N_NODES],
                         c0, c1, h, W2_l.T, W2_r.T, b2.reshape(1, D))
    return out


# speedup vs baseline: 3.3877x; 3.3877x over previous
import dataclasses
import functools

import jax
import jax.numpy as jnp
from jax import lax
from jax.experimental import pallas as pl
from jax.experimental.pallas import tpu as pltpu
from jax.experimental.pallas import tpu_sc as plsc

N_NODES = 10000
D = 128
N_EDGES = 320000

NC = 2
NS = 16
NW = NC * NS
L = 16
CHUNK = 128
NCH = 80
E_PAD = NW * CHUNK * NCH
EPW = NCH * CHUNK
SLAB = 8
RPT = 632
N_ACC = NS * RPT

_mesh = plsc.VectorSubcoreMesh(core_axis_name="c", subcore_axis_name="s")

_cp = pltpu.CompilerParams()
if "needs_layout_passes" in pltpu.CompilerParams.__dataclass_fields__:
    _cp = dataclasses.replace(_cp, needs_layout_passes=False)


def _sc_segment_sum(with_counts):
    out_type = [jax.ShapeDtypeStruct((NC * N_ACC, D), jnp.float32)]
    scratch = [
        pltpu.VMEM((SLAB, CHUNK), jnp.int32),
        pltpu.VMEM((SLAB, CHUNK), jnp.int32),
        pltpu.VMEM((CHUNK, D), jnp.float32),
        pltpu.VMEM_SHARED((N_ACC, D), jnp.float32),
        pltpu.SemaphoreType.DMA,
    ]
    if with_counts:
        out_type.append(jax.ShapeDtypeStruct((NW * N_ACC,), jnp.float32))
        scratch.append(pltpu.VMEM((N_ACC,), jnp.float32))

    def body(x_hbm, src_hbm, dst_hbm, zf_hbm, zc_hbm, sum_out, *rest):
        if with_counts:
            cnt_out, srcv, dstv, rows, accf, sem, cnt = rest
        else:
            (srcv, dstv, rows, accf, sem) = rest
        c = lax.axis_index("c")
        s = lax.axis_index("s")
        w = c * NS + s

        pltpu.sync_copy(zf_hbm.at[pl.ds(s * RPT, RPT)],
                        accf.at[pl.ds(s * RPT, RPT)])
        if with_counts:
            pltpu.sync_copy(zc_hbm, cnt)
        plsc.subcore_barrier()

        one = jnp.ones((L,), jnp.float32)

        @pl.loop(0, NCH // SLAB)
        def _(g):
            row0 = w * NCH + g * SLAB
            pltpu.sync_copy(src_hbm.at[pl.ds(row0, SLAB)], srcv)
            pltpu.sync_copy(dst_hbm.at[pl.ds(row0, SLAB)], dstv)
            for j in range(SLAB):
                pltpu.async_copy(x_hbm.at[srcv.at[j]], rows, sem).wait()
                pltpu.sync_copy(rows, accf.at[dstv.at[j]], add=True)
                if with_counts:
                    for k in range(CHUNK // L):
                        idx = dstv[j, pl.ds(k * L, L)]
                        plsc.addupdate_scatter(cnt, [idx], one)

        plsc.subcore_barrier()
        pltpu.sync_copy(accf.at[pl.ds(s * RPT, RPT)],
                        sum_out.at[pl.ds(c * N_ACC + s * RPT, RPT)])
        if with_counts:
            pltpu.sync_copy(cnt, cnt_out.at[pl.ds(w * N_ACC, N_ACC)])

    return pl.kernel(body, out_type=out_type, mesh=_mesh,
                     compiler_params=_cp, scratch_types=scratch)


_sc_sum_counts = _sc_segment_sum(True)
_sc_sum = _sc_segment_sum(False)

def _dense_body(act, p0, p1, c, x, wl, wr, b, o):
    cnt = jnp.sum(c[...], axis=0).reshape(N_ACC, 1)
    mean = (p0[...] + p1[...]) / jnp.maximum(cnt, 1.0)
    y = (jnp.dot(mean, wl[...], preferred_element_type=jnp.float32,
                 precision=lax.Precision.HIGHEST)
         + jnp.dot(x[...], wr[...], preferred_element_type=jnp.float32,
                   precision=lax.Precision.HIGHEST)
         + b[...])
    o[...] = jax.nn.relu(y) if act == "relu" else jax.nn.sigmoid(y)


def _dense(act):
    row_spec = pl.BlockSpec((N_ACC, D), lambda: (0, 0))
    cnt_spec = pl.BlockSpec((NW, N_ACC), lambda: (0, 0))
    full_spec = pl.BlockSpec((D, D), lambda: (0, 0))
    b_spec = pl.BlockSpec((1, D), lambda: (0, 0))
    return pl.pallas_call(
        functools.partial(_dense_body, act),
        in_specs=[row_spec, row_spec, cnt_spec, row_spec,
                  full_spec, full_spec, b_spec],
        out_specs=row_spec,
        out_shape=jax.ShapeDtypeStruct((N_ACC, D), jnp.float32),
    )


_dense_relu = _dense("relu")
_dense_sigmoid = _dense("sigmoid")


def kernel(x, edge_index, W1_l, W1_r, b1, W2_l, W2_r, b2):
    src = edge_index[0].astype(jnp.int32)
    dst = edge_index[1].astype(jnp.int32)
    pad = E_PAD - N_EDGES
    src_p = jnp.concatenate([src, jnp.zeros((pad,), jnp.int32)])
    dst_p = jnp.concatenate([dst, jnp.full((pad,), N_NODES, jnp.int32)])
    src_p = src_p.reshape(NW * NCH, CHUNK)
    dst_p = dst_p.reshape(NW * NCH, CHUNK)

    zf = jnp.zeros((N_ACC, D), jnp.float32)
    zc = jnp.zeros((N_ACC,), jnp.float32)

    def pad_rows(a):
        return jnp.concatenate(
            [a, jnp.zeros((N_ACC - a.shape[0], D), a.dtype)])

    x_pad = pad_rows(x)
    sums1, cnts = _sc_sum_counts(x_pad, src_p, dst_p, zf, zc)
    cnt2d = cnts.reshape(NW, N_ACC)
    h = _dense_relu(sums1[:N_ACC], sums1[N_ACC:],
                    cnt2d, x_pad, W1_l.T, W1_r.T, b1.reshape(1, D))
    (sums2,) = _sc_sum(h, src_p, dst_p, zf, zc)
    out = _dense_sigmoid(sums2[:N_ACC], sums2[N_ACC:],
                         cnt2d, h, W2_l.T, W2_r.T, b2.reshape(1, D))
    return out[:N_NODES]


# speedup vs baseline: 3.6861x; 1.0881x over previous
import dataclasses
import functools

import jax
import jax.numpy as jnp
from jax import lax
from jax.experimental import pallas as pl
from jax.experimental.pallas import tpu as pltpu
from jax.experimental.pallas import tpu_sc as plsc

N_NODES = 10000
D = 128
N_EDGES = 320000

NC = 2
NS = 16
NW = NC * NS
L = 16
CHUNK = 128
NCH = 80
E_PAD = NW * CHUNK * NCH
EPW = NCH * CHUNK
SLAB = 8
RPT = 632
N_ACC = NS * RPT

_mesh = plsc.VectorSubcoreMesh(core_axis_name="c", subcore_axis_name="s")

_cp = pltpu.CompilerParams()
if "needs_layout_passes" in pltpu.CompilerParams.__dataclass_fields__:
    _cp = dataclasses.replace(_cp, needs_layout_passes=False)


def _sc_segment_sum(with_counts):
    out_type = [jax.ShapeDtypeStruct((NC * N_ACC, D), jnp.float32)]
    scratch = [
        pltpu.VMEM((2, SLAB, CHUNK), jnp.int32),
        pltpu.VMEM((2, SLAB, CHUNK), jnp.int32),
        pltpu.VMEM((2, CHUNK, D), jnp.float32),
        pltpu.VMEM_SHARED((N_ACC, D), jnp.float32),
        pltpu.SemaphoreType.DMA,
        pltpu.SemaphoreType.DMA,
        pltpu.SemaphoreType.DMA,
        pltpu.SemaphoreType.DMA,
    ]
    if with_counts:
        out_type.append(jax.ShapeDtypeStruct((NW * N_ACC,), jnp.float32))
        scratch.append(pltpu.VMEM((N_ACC,), jnp.float32))

    def body(x_hbm, src_hbm, dst_hbm, zf_hbm, zc_hbm, sum_out, *rest):
        if with_counts:
            cnt_out, srcv, dstv, rows, accf, gsem0, gsem1, ssem, lsem, cnt = rest
        else:
            (srcv, dstv, rows, accf, gsem0, gsem1, ssem, lsem) = rest
        gsem = (gsem0, gsem1)
        c = lax.axis_index("c")
        s = lax.axis_index("s")
        w = c * NS + s
        base = w * NCH

        pltpu.sync_copy(zf_hbm.at[pl.ds(s * RPT, RPT)],
                        accf.at[pl.ds(s * RPT, RPT)])
        if with_counts:
            pltpu.sync_copy(zc_hbm, cnt)
        plsc.subcore_barrier()

        one = jnp.ones((L,), jnp.float32)

        def gather_fire(idx_row, buf_i):
            pltpu.async_copy(x_hbm.at[idx_row], rows.at[buf_i], gsem[buf_i])

        def gather_wait(buf_i):
            pltpu.make_async_copy(x_hbm.at[srcv.at[0, 0]], rows.at[buf_i],
                                  gsem[buf_i]).wait()

        pltpu.sync_copy(src_hbm.at[pl.ds(base, SLAB)], srcv.at[0])
        pltpu.sync_copy(dst_hbm.at[pl.ds(base, SLAB)], dstv.at[0])
        gather_fire(srcv.at[0, 0], 0)

        @pl.loop(0, NCH // SLAB // 2)
        def _(t):
            for ph in range(2):
                sp, so = ph, 1 - ph
                g = 2 * t + ph
                r0 = base + (g + 1) * SLAB
                ld0 = pltpu.async_copy(src_hbm.at[pl.ds(r0, SLAB)],
                                       srcv.at[so], lsem)
                ld1 = pltpu.async_copy(dst_hbm.at[pl.ds(r0, SLAB)],
                                       dstv.at[so], lsem)
                for j in range(SLAB):
                    q = ph * SLAB + j
                    b = q % 2
                    gather_wait(b)
                    scd = pltpu.async_copy(rows.at[b],
                                           accf.at[dstv.at[sp, j]],
                                           ssem, add=True)
                    if j < SLAB - 1:
                        gather_fire(srcv.at[sp, j + 1], 1 - b)
                    else:
                        ld0.wait()
                        ld1.wait()
                        gather_fire(srcv.at[so, 0], 1 - b)
                    if with_counts:
                        for k in range(CHUNK // L):
                            idx = dstv[sp, j, pl.ds(k * L, L)]
                            plsc.addupdate_scatter(cnt, [idx], one)
                    scd.wait()

        gather_wait(0)
        plsc.subcore_barrier()
        pltpu.sync_copy(accf.at[pl.ds(s * RPT, RPT)],
                        sum_out.at[pl.ds(c * N_ACC + s * RPT, RPT)])
        if with_counts:
            pltpu.sync_copy(cnt, cnt_out.at[pl.ds(w * N_ACC, N_ACC)])

    return pl.kernel(body, out_type=out_type, mesh=_mesh,
                     compiler_params=_cp, scratch_types=scratch)


_sc_sum_counts = _sc_segment_sum(True)
_sc_sum = _sc_segment_sum(False)

def _dense_body(act, p0, p1, c, x, wl, wr, b, o):
    cnt = jnp.sum(c[...], axis=0).reshape(N_ACC, 1)
    mean = (p0[...] + p1[...]) / jnp.maximum(cnt, 1.0)
    y = (jnp.dot(mean, wl[...], preferred_element_type=jnp.float32,
                 precision=lax.Precision.HIGHEST)
         + jnp.dot(x[...], wr[...], preferred_element_type=jnp.float32,
                   precision=lax.Precision.HIGHEST)
         + b[...])
    o[...] = jax.nn.relu(y) if act == "relu" else jax.nn.sigmoid(y)


def _dense(act):
    row_spec = pl.BlockSpec((N_ACC, D), lambda: (0, 0))
    cnt_spec = pl.BlockSpec((NW, N_ACC), lambda: (0, 0))
    full_spec = pl.BlockSpec((D, D), lambda: (0, 0))
    b_spec = pl.BlockSpec((1, D), lambda: (0, 0))
    return pl.pallas_call(
        functools.partial(_dense_body, act),
        in_specs=[row_spec, row_spec, cnt_spec, row_spec,
                  full_spec, full_spec, b_spec],
        out_specs=row_spec,
        out_shape=jax.ShapeDtypeStruct((N_ACC, D), jnp.float32),
    )


_dense_relu = _dense("relu")
_dense_sigmoid = _dense("sigmoid")


def kernel(x, edge_index, W1_l, W1_r, b1, W2_l, W2_r, b2):
    src = edge_index[0].astype(jnp.int32)
    dst = edge_index[1].astype(jnp.int32)
    pad = E_PAD - N_EDGES
    extra = SLAB * CHUNK
    src_p = jnp.concatenate([src, jnp.zeros((pad + extra,), jnp.int32)])
    dst_p = jnp.concatenate([dst, jnp.full((pad,), N_NODES, jnp.int32),
                             jnp.zeros((extra,), jnp.int32)])
    src_p = src_p.reshape(NW * NCH + SLAB, CHUNK)
    dst_p = dst_p.reshape(NW * NCH + SLAB, CHUNK)

    zf = jnp.zeros((N_ACC, D), jnp.float32)
    zc = jnp.zeros((N_ACC,), jnp.float32)

    def pad_rows(a):
        return jnp.concatenate(
            [a, jnp.zeros((N_ACC - a.shape[0], D), a.dtype)])

    x_pad = pad_rows(x)
    sums1, cnts = _sc_sum_counts(x_pad, src_p, dst_p, zf, zc)
    cnt2d = cnts.reshape(NW, N_ACC)
    h = _dense_relu(sums1[:N_ACC], sums1[N_ACC:],
                    cnt2d, x_pad, W1_l.T, W1_r.T, b1.reshape(1, D))
    (sums2,) = _sc_sum(h, src_p, dst_p, zf, zc)
    out = _dense_sigmoid(sums2[:N_ACC], sums2[N_ACC:],
                         cnt2d, h, W2_l.T, W2_r.T, b2.reshape(1, D))
    return out[:N_NODES]
